# baseline (speedup 1.0000x reference)
import functools

import jax
import jax.numpy as jnp
from jax import lax
from jax.experimental import pallas as pl
from jax.experimental.pallas import tpu as pltpu
from jax.experimental.pallas import tpu_sc as plsc

_MAXGRAD = 2.0
_B = 2
_D = 128
_D2 = _D * _D
_D3 = _D * _D * _D
_N = _B * _D3


def _logistic(x):
    c = _MAXGRAD
    return c / (1.0 + (c - 1.0) * jnp.exp(-x))


def _grids_body(d0r, d1r, d2r, s0r, n0r, s1r, n1r, s2r, n2r):
    f32 = jnp.float32
    r = lax.broadcasted_iota(jnp.int32, (_D, _D), 0)
    cidx = lax.broadcasted_iota(jnp.int32, (_D, _D), 1)
    ltri = (cidx <= r).astype(f32)
    utri = (r <= cidx).astype(f32)

    def mm(a, b):
        return lax.dot_general(
            a, b, (((1,), (0,)), ((), ())),
            precision=lax.Precision.HIGHEST,
            preferred_element_type=f32)

    a = _logistic(d0r[0])
    s = mm(ltri, a.reshape(_D, -1)).reshape(a.shape)
    first = s[0:1]
    last = s[_D - 1:_D]
    n = (s - first) * ((_D - 1.0) / (last - first + 1e-7))
    s0r[...] = (s - 1.0)[None]
    n0r[...] = n[None]

    a = _logistic(d1r[0])
    for i in range(a.shape[0]):
        s = mm(ltri, a[i])
        first = s[0:1, :]
        last = s[_D - 1:_D, :]
        n = (s - first) * ((_D - 1.0) / (last - first + 1e-7))
        s1r[0, i] = s - 1.0
        n1r[0, i] = n

    a = _logistic(d2r[0])
    for i in range(a.shape[0]):
        s = mm(a[i], utri)
        first = s[:, 0:1]
        last = s[:, _D - 1:_D]
        n = (s - first) * ((_D - 1.0) / (last - first + 1e-7))
        s2r[0, i] = s - 1.0
        n2r[0, i] = n


def _grids(d0, d1, d2, interpret=False):
    xspec = pl.BlockSpec((1, _D, 16, _D), lambda b, j: (b, 0, j, 0))
    yspec = pl.BlockSpec((1, 16, _D, _D), lambda b, j: (b, j, 0, 0))
    shp = jax.ShapeDtypeStruct((_B, _D, _D, _D), jnp.float32)
    return pl.pallas_call(
        _grids_body,
        grid=(_B, _D // 16),
        in_specs=[xspec, yspec, yspec],
        out_specs=[xspec, xspec,
                   yspec, yspec,
                   yspec, yspec],
        out_shape=[shp] * 6,
        interpret=interpret,
    )(d0, d1, d2)


_TAB = 64 * _D2


def _pp_body(cur_r, o_r):
    a = cur_r[0].reshape(8, 2, _D, _D)
    lo = lax.bitcast_convert_type(
        a[:, 0].astype(jnp.bfloat16), jnp.uint16).astype(jnp.uint32)
    hi = lax.bitcast_convert_type(
        a[:, 1].astype(jnp.bfloat16), jnp.uint16).astype(jnp.uint32)
    w = lax.bitcast_convert_type(lo | (hi << 16), jnp.int32)
    o_r[...] = w.reshape(8 * _D, _D)[None]


def _pppack(imt):
    return pl.pallas_call(
        _pp_body,
        grid=(_B, _D // 16),
        in_specs=[pl.BlockSpec((1, 16, _D, _D), lambda b, z: (b, z, 0, 0))],
        out_specs=pl.BlockSpec((1, 8 * _D, _D), lambda b, z: (b, z, 0)),
        out_shape=jax.ShapeDtypeStruct((_B, 64 * _D, _D), jnp.int32),
    )(imt).reshape(_B * _TAB)


def _pack3_body(ar, br, cr, o_r):
    m = lax.broadcasted_iota(jnp.int32, (_D, _D * 3), 0)
    n = lax.broadcasted_iota(jnp.int32, (_D, _D * 3), 1)
    acc = None
    for c, r in enumerate((ar, br, cr)):
        pc = (n == 3 * m + c).astype(jnp.float32)
        d = lax.dot_general(
            r[0].reshape(16 * _D, _D), pc, (((1,), (0,)), ((), ())),
            precision=lax.Precision.HIGHEST,
            preferred_element_type=jnp.float32)
        acc = d if acc is None else acc + d
    o_r[...] = acc.reshape(16, _D, _D * 3)[None]


def _pack3(a, b, c):
    spec = pl.BlockSpec((1, 16, _D, _D), lambda bi, j: (bi, j, 0, 0))
    return pl.pallas_call(
        _pack3_body,
        grid=(_B, _D // 16),
        in_specs=[spec, spec, spec],
        out_specs=pl.BlockSpec((1, 16, _D, _D * 3),
                               lambda bi, j: (bi, j, 0, 0)),
        out_shape=jax.ShapeDtypeStruct((_B, _D, _D, _D * 3), jnp.float32),
    )(a, b, c).reshape(_B, _D, _D, _D, 3)


_NW = 32
_NPW = _N // _NW
_CK = 1024
_NCHUNK = _NPW // _CK
_ROWS = _CK // _D
_STG = _TAB // 16
_STH = 8192


def _resample_body(inv, pp, cx, cy, cz, out, *sc):
    seta = sc[0:10]
    setb = sc[10:20]
    stb = sc[20]
    tab = sc[21]
    sem_a, sem_b, sem_oa, sem_ob, sem_ca, sem_cb = sc[22:28]
    cid = lax.axis_index("c")
    sid = lax.axis_index("s")
    base_pt = (cid * 16 + sid) * _NPW

    for h in range(_STG // _STH):
        soff = sid * _STG + h * _STH
        pltpu.sync_copy(pp.at[pl.ds(cid * _TAB + soff, _STH)], stb)
        pltpu.sync_copy(stb, tab.at[pl.ds(soff, _STH)])
    plsc.subcore_barrier()

    def mkset(bufs, gsem, osem, csem):
        cxb, cyb, czb, xdb, ydb, zdb, pob, outb = bufs[0:8]
        ii = bufs[8]
        vv = bufs[9]

        def fire_coords(t):
            off = base_pt + t * _CK
            pltpu.async_copy(cx.at[pl.ds(off, _CK)], cxb, csem)
            pltpu.async_copy(cy.at[pl.ds(off, _CK)], cyb, csem)
            pltpu.async_copy(cz.at[pl.ds(off, _CK)], czb, csem)

        def wait_prep_fire(t):
            for buf in (cxb, cyb, czb):
                pltpu.make_async_copy(
                    cx.at[pl.ds(base_pt, _CK)], buf, csem).wait()

            lane = lax.broadcasted_iota(jnp.int32, (16,), 0)

            def prep(i):
                sl = pl.ds(i * 16, 16)
                if inv:
                    pb = (sid * _NPW + t * _CK + i * 16) + lane
                    xr = (2 * (pb >> 14)).astype(jnp.float32) - cxb[sl]
                    yr = (2 * ((pb >> 7) & 127)).astype(jnp.float32) - cyb[sl]
                    zr = (2 * (pb & 127)).astype(jnp.float32) - czb[sl]
                else:
                    xr = cxb[sl]
                    yr = cyb[sl]
                    zr = czb[sl]
                x = jnp.clip(xr, 0.0, _D - 1.0)
                y = jnp.clip(yr, 0.0, _D - 1.0)
                z = jnp.clip(zr, 0.0, _D - 1.0)
                x0 = jnp.minimum(x.astype(jnp.int32), _D - 2)
                y0 = jnp.minimum(y.astype(jnp.int32), _D - 2)
                z0 = jnp.minimum(z.astype(jnp.int32), _D - 2)
                xdb[sl] = x - x0.astype(jnp.float32)
                ydb[sl] = y - y0.astype(jnp.float32)
                zdb[sl] = z - z0.astype(jnp.float32)
                podd = z0 & 1
                pob[sl] = podd
                v = (z0 >> 1) * _D2 + y0 * _D + x0
                vb = v + podd * _D2
                o = i * 16
                ii[pl.ds(o, 16)] = v
                ii[pl.ds(_CK + o, 16)] = vb
                ii[pl.ds(2 * _CK + o, 16)] = v + 1
                ii[pl.ds(3 * _CK + o, 16)] = vb + 1
                ii[pl.ds(4 * _CK + o, 16)] = v + _D
                ii[pl.ds(5 * _CK + o, 16)] = vb + _D
                ii[pl.ds(6 * _CK + o, 16)] = v + _D + 1
                ii[pl.ds(7 * _CK + o, 16)] = vb + _D + 1

            plsc.parallel_loop(0, _CK // 16, unroll=4)(prep)
            pltpu.async_copy(tab.at[ii], vv, gsem)

        def blend_out(t):
            pltpu.make_async_copy(pp.at[pl.ds(0, 8 * _CK)], vv, gsem).wait()

            @pl.when(t >= 2)
            def _():
                pltpu.make_async_copy(
                    out.at[pl.ds(base_pt, _CK)], outb, osem).wait()

            mask = jnp.int32(-65536)

            def blend(i):
                sl = pl.ds(i * 16, 16)
                xd = xdb[sl]
                yd = ydb[sl]
                zd = zdb[sl]
                odd = pob[sl]
                sh0 = (1 - odd) << 4
                sh1 = odd << 4

                o = i * 16

                def zmix(ga, gb):
                    wa = vv[pl.ds(ga * _CK + o, 16)]
                    wb = vv[pl.ds(gb * _CK + o, 16)]
                    vz0 = lax.bitcast_convert_type((wa << sh0) & mask,
                                                   jnp.float32)
                    vz1 = lax.bitcast_convert_type((wb << sh1) & mask,
                                                   jnp.float32)
                    return vz0 + zd * (vz1 - vz0)

                c00 = zmix(0, 1)
                c01 = zmix(2, 3)
                c10 = zmix(4, 5)
                c11 = zmix(6, 7)
                r0 = c00 + xd * (c01 - c00)
                r1 = c10 + xd * (c11 - c10)
                outb[sl] = r0 + yd * (r1 - r0)

            plsc.parallel_loop(0, _CK // 16, unroll=4)(blend)
            pltpu.async_copy(outb, out.at[pl.ds(base_pt + t * _CK, _CK)],
                             osem)

        def drain_out():
            pltpu.make_async_copy(
                out.at[pl.ds(base_pt, _CK)], outb, osem).wait()

        return fire_coords, wait_prep_fire, blend_out, drain_out

    fc_a, wpf_a, blo_a, dr_a = mkset(seta, sem_a, sem_oa, sem_ca)
    fc_b, wpf_b, blo_b, dr_b = mkset(setb, sem_b, sem_ob, sem_cb)

    fc_a(0)
    fc_b(1)
    wpf_a(0)

    def pair(u, _):
        t0 = u * 2
        fc_a(t0 + 2)
        wpf_b(t0 + 1)
        blo_a(t0)
        fc_b(t0 + 3)
        wpf_a(t0 + 2)
        blo_b(t0 + 1)
        return 0

    lax.fori_loop(0, _NCHUNK // 2 - 1, pair, 0)
    wpf_b(_NCHUNK - 1)
    blo_a(_NCHUNK - 2)
    blo_b(_NCHUNK - 1)
    dr_a()
    dr_b()


@functools.partial(jax.jit, static_argnames=("inv", "interpret"))
def _resample(pp, cx, cy, cz, inv=False, interpret=False):
    mesh = plsc.VectorSubcoreMesh(
        core_axis_name="c", subcore_axis_name="s", num_cores=2)
    bufset = ([pltpu.VMEM((_CK,), jnp.float32)] * 6
              + [pltpu.VMEM((_CK,), jnp.int32)]
              + [pltpu.VMEM((_CK,), jnp.float32)]
              + [pltpu.VMEM((8 * _CK,), jnp.int32)] * 2)
    return pl.kernel(
        functools.partial(_resample_body, inv),
        out_type=jax.ShapeDtypeStruct((_N,), jnp.float32),
        mesh=mesh,
        scratch_types=(
            bufset + bufset + [
                pltpu.VMEM((_STH,), jnp.int32),
                pltpu.VMEM_SHARED((_TAB,), jnp.int32),
                pltpu.SemaphoreType.DMA,
                pltpu.SemaphoreType.DMA,
                pltpu.SemaphoreType.DMA,
                pltpu.SemaphoreType.DMA,
                pltpu.SemaphoreType.DMA,
                pltpu.SemaphoreType.DMA,
            ]),
        interpret=interpret,
    )(pp, cx, cy, cz)


def kernel(mov, ref, defgrad):
    d0 = defgrad[..., 0]
    d1 = defgrad[..., 1]
    d2 = defgrad[..., 2]
    s0, n0, s1, n1, s2, n2 = _grids(d0, d1, d2)

    norm = jnp.stack([n0, n1, n2], axis=-1)
    f32 = jnp.float32
    oshape = (_B, _D, _D, _D, 3)
    base = (lax.broadcasted_iota(f32, oshape, 1)
            * (lax.broadcasted_iota(jnp.int32, oshape, 4) == 0)
            + lax.broadcasted_iota(f32, oshape, 2)
            * (lax.broadcasted_iota(jnp.int32, oshape, 4) == 1)
            + lax.broadcasted_iota(f32, oshape, 3)
            * (lax.broadcasted_iota(jnp.int32, oshape, 4) == 2))
    inverse = 2.0 * base - norm

    mov_t = jnp.transpose(mov.reshape(_B, _D, _D, _D), (0, 3, 2, 1))
    ref_t = jnp.transpose(ref.reshape(_B, _D, _D, _D), (0, 3, 2, 1))
    mov_pp = _pppack(mov_t)
    ref_pp = _pppack(ref_t)

    mov_def = _resample(mov_pp, s0.reshape(-1), s1.reshape(-1),
                        s2.reshape(-1))
    ref_def = _resample(ref_pp, n0.reshape(-1), n1.reshape(-1),
                        n2.reshape(-1), inv=True)

    out_shape = (_B, _D, _D, _D, 1)
    return (mov_def.reshape(out_shape), ref_def.reshape(out_shape),
            norm, inverse)


# speedup vs baseline: 1.0120x; 1.0120x over previous
import functools

import jax
import jax.numpy as jnp
from jax import lax
from jax.experimental import pallas as pl
from jax.experimental.pallas import tpu as pltpu
from jax.experimental.pallas import tpu_sc as plsc

_MAXGRAD = 2.0
_B = 2
_D = 128
_D2 = _D * _D
_D3 = _D * _D * _D
_N = _B * _D3


def _logistic(x):
    c = _MAXGRAD
    return c / (1.0 + (c - 1.0) * jnp.exp(-x))


def _grids_body(d0r, d1r, d2r, s0r, n0r, s1r, n1r, s2r, n2r):
    f32 = jnp.float32
    r = lax.broadcasted_iota(jnp.int32, (_D, _D), 0)
    cidx = lax.broadcasted_iota(jnp.int32, (_D, _D), 1)
    ltri = (cidx <= r).astype(f32)
    utri = (r <= cidx).astype(f32)

    bf16 = jnp.bfloat16

    def dot(a, b):
        return lax.dot_general(
            a, b, (((1,), (0,)), ((), ())), preferred_element_type=f32)

    def mm(tri, a):
        ah = a.astype(bf16)
        al = (a - ah.astype(f32)).astype(bf16)
        return dot(tri.astype(bf16), ah) + dot(tri.astype(bf16), al)

    def mmr(a, tri):
        ah = a.astype(bf16)
        al = (a - ah.astype(f32)).astype(bf16)
        return dot(ah, tri.astype(bf16)) + dot(al, tri.astype(bf16))

    a = _logistic(d0r[0])
    s = mm(ltri, a.reshape(_D, -1)).reshape(a.shape)
    first = s[0:1]
    last = s[_D - 1:_D]
    n = (s - first) * ((_D - 1.0) / (last - first + 1e-7))
    s0r[...] = (s - 1.0)[None]
    n0r[...] = n[None]

    a = _logistic(d1r[0])
    for i in range(a.shape[0]):
        s = mm(ltri, a[i])
        first = s[0:1, :]
        last = s[_D - 1:_D, :]
        n = (s - first) * ((_D - 1.0) / (last - first + 1e-7))
        s1r[0, i] = s - 1.0
        n1r[0, i] = n

    a = _logistic(d2r[0])
    for i in range(a.shape[0]):
        s = mmr(a[i], utri)
        first = s[:, 0:1]
        last = s[:, _D - 1:_D]
        n = (s - first) * ((_D - 1.0) / (last - first + 1e-7))
        s2r[0, i] = s - 1.0
        n2r[0, i] = n


def _grids(d0, d1, d2, interpret=False):
    xspec = pl.BlockSpec((1, _D, 16, _D), lambda b, j: (b, 0, j, 0))
    yspec = pl.BlockSpec((1, 16, _D, _D), lambda b, j: (b, j, 0, 0))
    shp = jax.ShapeDtypeStruct((_B, _D, _D, _D), jnp.float32)
    return pl.pallas_call(
        _grids_body,
        grid=(_B, _D // 16),
        in_specs=[xspec, yspec, yspec],
        out_specs=[xspec, xspec,
                   yspec, yspec,
                   yspec, yspec],
        out_shape=[shp] * 6,
        interpret=interpret,
    )(d0, d1, d2)


_TAB = 64 * _D2


def _pp_body(cur_r, o_r):
    a = cur_r[0].reshape(8, 2, _D, _D)
    lo = lax.bitcast_convert_type(
        a[:, 0].astype(jnp.bfloat16), jnp.uint16).astype(jnp.uint32)
    hi = lax.bitcast_convert_type(
        a[:, 1].astype(jnp.bfloat16), jnp.uint16).astype(jnp.uint32)
    w = lax.bitcast_convert_type(lo | (hi << 16), jnp.int32)
    o_r[...] = w.reshape(8 * _D, _D)[None]


def _pppack(imt):
    return pl.pallas_call(
        _pp_body,
        grid=(_B, _D // 16),
        in_specs=[pl.BlockSpec((1, 16, _D, _D), lambda b, z: (b, z, 0, 0))],
        out_specs=pl.BlockSpec((1, 8 * _D, _D), lambda b, z: (b, z, 0)),
        out_shape=jax.ShapeDtypeStruct((_B, 64 * _D, _D), jnp.int32),
    )(imt).reshape(_B * _TAB)


def _pack3_body(ar, br, cr, o_r):
    m = lax.broadcasted_iota(jnp.int32, (_D, _D * 3), 0)
    n = lax.broadcasted_iota(jnp.int32, (_D, _D * 3), 1)
    acc = None
    for c, r in enumerate((ar, br, cr)):
        pc = (n == 3 * m + c).astype(jnp.float32)
        d = lax.dot_general(
            r[0].reshape(16 * _D, _D), pc, (((1,), (0,)), ((), ())),
            precision=lax.Precision.HIGHEST,
            preferred_element_type=jnp.float32)
        acc = d if acc is None else acc + d
    o_r[...] = acc.reshape(16, _D, _D * 3)[None]


def _pack3(a, b, c):
    spec = pl.BlockSpec((1, 16, _D, _D), lambda bi, j: (bi, j, 0, 0))
    return pl.pallas_call(
        _pack3_body,
        grid=(_B, _D // 16),
        in_specs=[spec, spec, spec],
        out_specs=pl.BlockSpec((1, 16, _D, _D * 3),
                               lambda bi, j: (bi, j, 0, 0)),
        out_shape=jax.ShapeDtypeStruct((_B, _D, _D, _D * 3), jnp.float32),
    )(a, b, c).reshape(_B, _D, _D, _D, 3)


_NW = 32
_NPW = _N // _NW
_CK = 1024
_NCHUNK = _NPW // _CK
_ROWS = _CK // _D
_STG = _TAB // 16
_STH = 8192


def _resample_body(inv, pp, cx, cy, cz, out, *sc):
    seta = sc[0:10]
    setb = sc[10:20]
    stb = sc[20]
    tab = sc[21]
    sem_a, sem_b, sem_oa, sem_ob, sem_ca, sem_cb = sc[22:28]
    cid = lax.axis_index("c")
    sid = lax.axis_index("s")
    base_pt = (cid * 16 + sid) * _NPW

    for h in range(_STG // _STH):
        soff = sid * _STG + h * _STH
        pltpu.sync_copy(pp.at[pl.ds(cid * _TAB + soff, _STH)], stb)
        pltpu.sync_copy(stb, tab.at[pl.ds(soff, _STH)])
    plsc.subcore_barrier()

    def mkset(bufs, gsem, osem, csem):
        cxb, cyb, czb, xdb, ydb, zdb, pob, outb = bufs[0:8]
        ii = bufs[8]
        vv = bufs[9]

        def fire_coords(t):
            off = base_pt + t * _CK
            pltpu.async_copy(cx.at[pl.ds(off, _CK)], cxb, csem)
            pltpu.async_copy(cy.at[pl.ds(off, _CK)], cyb, csem)
            pltpu.async_copy(cz.at[pl.ds(off, _CK)], czb, csem)

        def wait_prep_fire(t):
            for buf in (cxb, cyb, czb):
                pltpu.make_async_copy(
                    cx.at[pl.ds(base_pt, _CK)], buf, csem).wait()

            lane = lax.broadcasted_iota(jnp.int32, (16,), 0)

            def prep(i):
                sl = pl.ds(i * 16, 16)
                if inv:
                    pb = (sid * _NPW + t * _CK + i * 16) + lane
                    xr = (2 * (pb >> 14)).astype(jnp.float32) - cxb[sl]
                    yr = (2 * ((pb >> 7) & 127)).astype(jnp.float32) - cyb[sl]
                    zr = (2 * (pb & 127)).astype(jnp.float32) - czb[sl]
                else:
                    xr = cxb[sl]
                    yr = cyb[sl]
                    zr = czb[sl]
                x = jnp.clip(xr, 0.0, _D - 1.0)
                y = jnp.clip(yr, 0.0, _D - 1.0)
                z = jnp.clip(zr, 0.0, _D - 1.0)
                x0 = jnp.minimum(x.astype(jnp.int32), _D - 2)
                y0 = jnp.minimum(y.astype(jnp.int32), _D - 2)
                z0 = jnp.minimum(z.astype(jnp.int32), _D - 2)
                xdb[sl] = x - x0.astype(jnp.float32)
                ydb[sl] = y - y0.astype(jnp.float32)
                zdb[sl] = z - z0.astype(jnp.float32)
                podd = z0 & 1
                pob[sl] = podd
                v = (z0 >> 1) * _D2 + y0 * _D + x0
                vb = v + podd * _D2
                o = i * 16
                ii[pl.ds(o, 16)] = v
                ii[pl.ds(_CK + o, 16)] = vb
                ii[pl.ds(2 * _CK + o, 16)] = v + 1
                ii[pl.ds(3 * _CK + o, 16)] = vb + 1
                ii[pl.ds(4 * _CK + o, 16)] = v + _D
                ii[pl.ds(5 * _CK + o, 16)] = vb + _D
                ii[pl.ds(6 * _CK + o, 16)] = v + _D + 1
                ii[pl.ds(7 * _CK + o, 16)] = vb + _D + 1

            plsc.parallel_loop(0, _CK // 16, unroll=4)(prep)
            pltpu.async_copy(tab.at[ii], vv, gsem)

        def blend_out(t):
            pltpu.make_async_copy(pp.at[pl.ds(0, 8 * _CK)], vv, gsem).wait()

            @pl.when(t >= 2)
            def _():
                pltpu.make_async_copy(
                    out.at[pl.ds(base_pt, _CK)], outb, osem).wait()

            mask = jnp.int32(-65536)

            def blend(i):
                sl = pl.ds(i * 16, 16)
                xd = xdb[sl]
                yd = ydb[sl]
                zd = zdb[sl]
                odd = pob[sl]
                sh0 = (1 - odd) << 4
                sh1 = odd << 4

                o = i * 16

                def zmix(ga, gb):
                    wa = vv[pl.ds(ga * _CK + o, 16)]
                    wb = vv[pl.ds(gb * _CK + o, 16)]
                    vz0 = lax.bitcast_convert_type((wa << sh0) & mask,
                                                   jnp.float32)
                    vz1 = lax.bitcast_convert_type((wb << sh1) & mask,
                                                   jnp.float32)
                    return vz0 + zd * (vz1 - vz0)

                c00 = zmix(0, 1)
                c01 = zmix(2, 3)
                c10 = zmix(4, 5)
                c11 = zmix(6, 7)
                r0 = c00 + xd * (c01 - c00)
                r1 = c10 + xd * (c11 - c10)
                outb[sl] = r0 + yd * (r1 - r0)

            plsc.parallel_loop(0, _CK // 16, unroll=4)(blend)
            pltpu.async_copy(outb, out.at[pl.ds(base_pt + t * _CK, _CK)],
                             osem)

        def drain_out():
            pltpu.make_async_copy(
                out.at[pl.ds(base_pt, _CK)], outb, osem).wait()

        return fire_coords, wait_prep_fire, blend_out, drain_out

    fc_a, wpf_a, blo_a, dr_a = mkset(seta, sem_a, sem_oa, sem_ca)
    fc_b, wpf_b, blo_b, dr_b = mkset(setb, sem_b, sem_ob, sem_cb)

    fc_a(0)
    fc_b(1)
    wpf_a(0)

    def pair(u, _):
        t0 = u * 2
        fc_a(t0 + 2)
        wpf_b(t0 + 1)
        blo_a(t0)
        fc_b(t0 + 3)
        wpf_a(t0 + 2)
        blo_b(t0 + 1)
        return 0

    lax.fori_loop(0, _NCHUNK // 2 - 1, pair, 0)
    wpf_b(_NCHUNK - 1)
    blo_a(_NCHUNK - 2)
    blo_b(_NCHUNK - 1)
    dr_a()
    dr_b()


@functools.partial(jax.jit, static_argnames=("inv", "interpret"))
def _resample(pp, cx, cy, cz, inv=False, interpret=False):
    mesh = plsc.VectorSubcoreMesh(
        core_axis_name="c", subcore_axis_name="s", num_cores=2)
    bufset = ([pltpu.VMEM((_CK,), jnp.float32)] * 6
              + [pltpu.VMEM((_CK,), jnp.int32)]
              + [pltpu.VMEM((_CK,), jnp.float32)]
              + [pltpu.VMEM((8 * _CK,), jnp.int32)] * 2)
    return pl.kernel(
        functools.partial(_resample_body, inv),
        out_type=jax.ShapeDtypeStruct((_N,), jnp.float32),
        mesh=mesh,
        scratch_types=(
            bufset + bufset + [
                pltpu.VMEM((_STH,), jnp.int32),
                pltpu.VMEM_SHARED((_TAB,), jnp.int32),
                pltpu.SemaphoreType.DMA,
                pltpu.SemaphoreType.DMA,
                pltpu.SemaphoreType.DMA,
                pltpu.SemaphoreType.DMA,
                pltpu.SemaphoreType.DMA,
                pltpu.SemaphoreType.DMA,
            ]),
        interpret=interpret,
    )(pp, cx, cy, cz)


def kernel(mov, ref, defgrad):
    d0 = defgrad[..., 0]
    d1 = defgrad[..., 1]
    d2 = defgrad[..., 2]
    s0, n0, s1, n1, s2, n2 = _grids(d0, d1, d2)

    norm = jnp.stack([n0, n1, n2], axis=-1)
    f32 = jnp.float32
    oshape = (_B, _D, _D, _D, 3)
    base = (lax.broadcasted_iota(f32, oshape, 1)
            * (lax.broadcasted_iota(jnp.int32, oshape, 4) == 0)
            + lax.broadcasted_iota(f32, oshape, 2)
            * (lax.broadcasted_iota(jnp.int32, oshape, 4) == 1)
            + lax.broadcasted_iota(f32, oshape, 3)
            * (lax.broadcasted_iota(jnp.int32, oshape, 4) == 2))
    inverse = 2.0 * base - norm

    mov_t = jnp.transpose(mov.reshape(_B, _D, _D, _D), (0, 3, 2, 1))
    ref_t = jnp.transpose(ref.reshape(_B, _D, _D, _D), (0, 3, 2, 1))
    mov_pp = _pppack(mov_t)
    ref_pp = _pppack(ref_t)

    mov_def = _resample(mov_pp, s0.reshape(-1), s1.reshape(-1),
                        s2.reshape(-1))
    ref_def = _resample(ref_pp, n0.reshape(-1), n1.reshape(-1),
                        n2.reshape(-1), inv=True)

    out_shape = (_B, _D, _D, _D, 1)
    return (mov_def.reshape(out_shape), ref_def.reshape(out_shape),
            norm, inverse)
